# baseline (speedup 1.0000x reference)
import functools

import jax
import jax.numpy as jnp
from jax import lax
from jax.experimental import pallas as pl
from jax.experimental.pallas import tpu as pltpu
from jax.experimental.pallas import tpu_sc as plsc

T = 2048
D = 768
F = 2048
E = 8
K = 2

B = 256
P = T * K
P_PAD = P + E * B
G = P_PAD // B

NC = 2
NS = 16
NW = NC * NS


@functools.lru_cache(maxsize=None)
def _make_sc_row_gather(n_rows: int, chunk: int):
    n_per_w = n_rows // NW
    n_chunks = n_per_w // chunk
    assert n_per_w % chunk == 0 and chunk % 8 == 0

    mesh = plsc.VectorSubcoreMesh(core_axis_name="c", subcore_axis_name="s",
                                  num_cores=NC, num_subcores=NS)

    @functools.partial(
        pl.kernel,
        mesh=mesh,
        out_type=jax.ShapeDtypeStruct((n_rows, D), jnp.float32),
        scratch_types=[
            pltpu.VMEM((chunk,), jnp.int32),
            pltpu.VMEM((chunk, D), jnp.float32),
            pltpu.SemaphoreType.DMA,
        ],
    )
    def gather_kernel(table_hbm, idx_hbm, out_hbm, idx_v, rows_v, sem):
        wid = lax.axis_index("s") * NC + lax.axis_index("c")
        base = wid * n_per_w
        for c in range(n_chunks):
            off = base + c * chunk
            pltpu.sync_copy(idx_hbm.at[pl.ds(off, chunk)], idx_v)
            pltpu.async_copy(table_hbm.at[idx_v], rows_v, sem).wait()
            pltpu.sync_copy(rows_v, out_hbm.at[pl.ds(off, chunk)])

    return gather_kernel


def _ffn_body(blk_ref, eid_ref, x_ref, w0_ref, w1_ref, w2_ref, rw_ref, o_ref):
    s = pl.program_id(0)

    @pl.when(blk_ref[s] == s)
    def _():
        xb = x_ref[...]
        a = jnp.dot(xb, w0_ref[0], preferred_element_type=jnp.float32,
                    precision=lax.Precision.DEFAULT)
        b = jnp.dot(xb, w1_ref[0], preferred_element_type=jnp.float32,
                    precision=lax.Precision.DEFAULT)
        h = (a * jax.nn.sigmoid(a)) * b
        y = jnp.dot(h, w2_ref[0], preferred_element_type=jnp.float32,
                    precision=lax.Precision.DEFAULT)
        o_ref[...] = y * rw_ref[...]


_ffn_grid_spec = pltpu.PrefetchScalarGridSpec(
    num_scalar_prefetch=2,
    grid=(G,),
    in_specs=[
        pl.BlockSpec((B, D), lambda s, blk, eid: (blk[s], 0)),
        pl.BlockSpec((1, D, F), lambda s, blk, eid: (eid[s], 0, 0)),
        pl.BlockSpec((1, D, F), lambda s, blk, eid: (eid[s], 0, 0)),
        pl.BlockSpec((1, F, D), lambda s, blk, eid: (eid[s], 0, 0)),
        pl.BlockSpec((B, 1), lambda s, blk, eid: (blk[s], 0)),
    ],
    out_specs=pl.BlockSpec((B, D), lambda s, blk, eid: (blk[s], 0)),
)


def _routing_metadata(e_flat, rw_flat):
    order = jnp.argsort(e_flat, stable=True).astype(jnp.int32)
    onehot = (e_flat[:, None] == jnp.arange(E, dtype=jnp.int32)[None, :])
    cum = jnp.cumsum(onehot.astype(jnp.int32), axis=0)
    counts = cum[-1]
    rank = jnp.take_along_axis(cum, e_flat[:, None], axis=1)[:, 0] - 1
    csum = jnp.cumsum(counts)
    group_start = csum - counts
    padded_counts = ((counts + B - 1) // B) * B
    pcsum = jnp.cumsum(padded_counts)
    pad_start = pcsum - padded_counts
    pos = (jnp.take(pad_start, e_flat) + rank).astype(jnp.int32)
    q_ids = jnp.arange(P_PAD, dtype=jnp.int32)
    eq = jnp.searchsorted(pcsum, q_ids, side="right").astype(jnp.int32)
    eq = jnp.minimum(eq, E - 1)
    rq = q_ids - jnp.take(pad_start, eq)
    valid = rq < jnp.take(counts, eq)
    p_q = jnp.take(group_start, eq) + jnp.minimum(rq, jnp.take(counts, eq) - 1)
    slot_q = jnp.take(order, jnp.clip(p_q, 0, P - 1))
    tok_padded = jnp.where(valid, slot_q // K, q_ids % T)
    rw_padded = jnp.where(valid, jnp.take(rw_flat, slot_q), 0.0)

    nb = pcsum[-1] // B
    s_ids = jnp.arange(G, dtype=jnp.int32)
    blk = jnp.minimum(s_ids, nb - 1).astype(jnp.int32)
    eid = jnp.searchsorted(pcsum, blk * B, side="right").astype(jnp.int32)
    return tok_padded, rw_padded, pos, blk, eid


def kernel(x, w0, w1, w2, selected_experts, routing_weights):
    e_flat = selected_experts.reshape(P).astype(jnp.int32)
    rw_flat = routing_weights.reshape(P)
    tok_padded, rw_padded, pos, blk, eid = _routing_metadata(e_flat, rw_flat)

    x_padded = _make_sc_row_gather(P_PAD, 96)(x, tok_padded)

    y_scaled = pl.pallas_call(
        _ffn_body,
        grid_spec=_ffn_grid_spec,
        out_shape=jax.ShapeDtypeStruct((P_PAD, D), jnp.float32),
    )(blk, eid, x_padded, w0, w1, w2, rw_padded[:, None])

    out_flat = _make_sc_row_gather(P, 128)(y_scaled, pos)
    return out_flat.reshape(T, K, D)


# speedup vs baseline: 1.3736x; 1.3736x over previous
import functools

import jax
import jax.numpy as jnp
from jax import lax
from jax.experimental import pallas as pl
from jax.experimental.pallas import tpu as pltpu
from jax.experimental.pallas import tpu_sc as plsc

T = 2048
D = 768
F = 2048
E = 8
K = 2

B = 256
P = T * K
P_PAD = P + E * B
G = P_PAD // B

NC = 2
NS = 16
NW = NC * NS
TPW = T // NW


def _sc_mesh():
    return plsc.VectorSubcoreMesh(core_axis_name="c", subcore_axis_name="s",
                                  num_cores=NC, num_subcores=NS)


@functools.lru_cache(maxsize=None)
def _make_sc_dispatch():
    @functools.partial(
        pl.kernel,
        mesh=_sc_mesh(),
        out_type=jax.ShapeDtypeStruct((P_PAD, D), jnp.float32),
        scratch_types=[
            pltpu.VMEM((TPW,), jnp.int32),
            pltpu.VMEM((TPW,), jnp.int32),
            pltpu.VMEM((TPW, D), jnp.float32),
            pltpu.SemaphoreType.DMA,
        ],
    )
    def dispatch_kernel(x_hbm, pe_hbm, po_hbm, out_hbm, idxe_v, idxo_v,
                        rows_v, sem):
        wid = lax.axis_index("s") * NC + lax.axis_index("c")
        tb = wid * TPW
        pltpu.sync_copy(x_hbm.at[pl.ds(tb, TPW)], rows_v)
        pltpu.sync_copy(pe_hbm.at[pl.ds(tb, TPW)], idxe_v)
        pltpu.sync_copy(po_hbm.at[pl.ds(tb, TPW)], idxo_v)
        c1 = pltpu.async_copy(rows_v, out_hbm.at[idxe_v], sem)
        c2 = pltpu.async_copy(rows_v, out_hbm.at[idxo_v], sem)
        c1.wait()
        c2.wait()

    return dispatch_kernel


@functools.lru_cache(maxsize=None)
def _make_sc_combine(n_rows: int, chunk: int):
    n_per_w = n_rows // NW
    n_chunks = n_per_w // chunk
    assert n_per_w % chunk == 0 and chunk % 8 == 0

    @functools.partial(
        pl.kernel,
        mesh=_sc_mesh(),
        out_type=jax.ShapeDtypeStruct((n_rows, D), jnp.float32),
        scratch_types=[
            pltpu.VMEM((chunk,), jnp.int32),
            pltpu.VMEM((chunk, D), jnp.float32),
            pltpu.SemaphoreType.DMA,
        ],
    )
    def gather_kernel(table_hbm, idx_hbm, out_hbm, idx_v, rows_v, sem):
        wid = lax.axis_index("s") * NC + lax.axis_index("c")
        base = wid * n_per_w
        for c in range(n_chunks):
            off = base + c * chunk
            pltpu.sync_copy(idx_hbm.at[pl.ds(off, chunk)], idx_v)
            pltpu.async_copy(table_hbm.at[idx_v], rows_v, sem).wait()
            pltpu.sync_copy(rows_v, out_hbm.at[pl.ds(off, chunk)])

    return gather_kernel


def _ffn_body(blk_ref, eid_ref, x_ref, w0_ref, w1_ref, w2_ref, o_ref):
    s = pl.program_id(0)

    @pl.when(blk_ref[s] == s)
    def _():
        xb = x_ref[...]
        a = jnp.dot(xb, w0_ref[0], preferred_element_type=jnp.float32,
                    precision=lax.Precision.DEFAULT)
        b = jnp.dot(xb, w1_ref[0], preferred_element_type=jnp.float32,
                    precision=lax.Precision.DEFAULT)
        h = (a * jax.nn.sigmoid(a)) * b
        o_ref[...] = jnp.dot(h, w2_ref[0], preferred_element_type=jnp.float32,
                             precision=lax.Precision.DEFAULT)


_ffn_grid_spec = pltpu.PrefetchScalarGridSpec(
    num_scalar_prefetch=2,
    grid=(G,),
    in_specs=[
        pl.BlockSpec((B, D), lambda s, blk, eid: (blk[s], 0)),
        pl.BlockSpec((1, D, F), lambda s, blk, eid: (eid[s], 0, 0)),
        pl.BlockSpec((1, D, F), lambda s, blk, eid: (eid[s], 0, 0)),
        pl.BlockSpec((1, F, D), lambda s, blk, eid: (eid[s], 0, 0)),
    ],
    out_specs=pl.BlockSpec((B, D), lambda s, blk, eid: (blk[s], 0)),
)


def _routing_metadata(e2d):
    e_flat = e2d.reshape(P)
    oh = (e_flat[:, None] == jnp.arange(E, dtype=jnp.int32)[None, :])
    oh_b = oh.reshape(NW, P // NW, E).astype(jnp.float32)
    tril_fine = jnp.tril(jnp.ones((P // NW, P // NW), jnp.float32), k=-1)
    fine = jnp.einsum("ij,bjE->biE", tril_fine, oh_b,
                      precision=lax.Precision.HIGHEST)
    bs = oh_b.sum(axis=1)
    tril_coarse = jnp.tril(jnp.ones((NW, NW), jnp.float32), k=-1)
    coarse = tril_coarse @ bs
    rank = (fine + coarse[:, None, :]).reshape(P, E)
    rank = jnp.take_along_axis(rank, e_flat[:, None], axis=1)[:, 0]
    counts = bs.sum(axis=0)
    padded_counts = jnp.ceil(counts / B) * B
    pcsum = (jnp.tril(jnp.ones((E, E), jnp.float32)) @ padded_counts)
    pad_start = pcsum - padded_counts
    pos = (jnp.take(pad_start, e_flat) + rank).astype(jnp.int32)

    nb = (pcsum[E - 1] / B).astype(jnp.int32)
    s_ids = jnp.arange(G, dtype=jnp.int32)
    blk = jnp.minimum(s_ids, nb - 1)
    starts = (blk * B).astype(jnp.float32)
    eid = jnp.sum(pcsum[None, :] <= starts[:, None], axis=1).astype(jnp.int32)
    return pos.reshape(T, K), blk, eid


def kernel(x, w0, w1, w2, selected_experts, routing_weights):
    e2d = selected_experts.astype(jnp.int32)
    pos2d, blk, eid = _routing_metadata(e2d)

    x_padded = _make_sc_dispatch()(x, pos2d[:, 0], pos2d[:, 1])

    y = pl.pallas_call(
        _ffn_body,
        grid_spec=_ffn_grid_spec,
        out_shape=jax.ShapeDtypeStruct((P_PAD, D), jnp.float32),
    )(blk, eid, x_padded, w0, w1, w2)

    out_flat = _make_sc_combine(P, 128)(y, pos2d.reshape(P))
    out = out_flat.reshape(T, K, D) * routing_weights[:, :, None]
    return out


# speedup vs baseline: 1.3859x; 1.0089x over previous
import functools

import jax
import jax.numpy as jnp
from jax import lax
from jax.experimental import pallas as pl
from jax.experimental.pallas import tpu as pltpu
from jax.experimental.pallas import tpu_sc as plsc

T = 2048
D = 768
F = 2048
E = 8
K = 2

B = 256
P = T * K
P_PAD = P + E * B
G = P_PAD // B

NC = 2
NS = 16
NW = NC * NS
TPW = T // NW


def _sc_mesh():
    return plsc.VectorSubcoreMesh(core_axis_name="c", subcore_axis_name="s",
                                  num_cores=NC, num_subcores=NS)


@functools.lru_cache(maxsize=None)
def _make_sc_dispatch():
    @functools.partial(
        pl.kernel,
        mesh=_sc_mesh(),
        out_type=jax.ShapeDtypeStruct((P_PAD, D), jnp.float32),
        scratch_types=[
            pltpu.VMEM((TPW,), jnp.int32),
            pltpu.VMEM((TPW,), jnp.int32),
            pltpu.VMEM((TPW, D), jnp.float32),
            pltpu.SemaphoreType.DMA,
        ],
    )
    def dispatch_kernel(x_hbm, pe_hbm, po_hbm, out_hbm, idxe_v, idxo_v,
                        rows_v, sem):
        wid = lax.axis_index("s") * NC + lax.axis_index("c")
        tb = wid * TPW
        pltpu.sync_copy(x_hbm.at[pl.ds(tb, TPW)], rows_v)
        pltpu.sync_copy(pe_hbm.at[pl.ds(tb, TPW)], idxe_v)
        pltpu.sync_copy(po_hbm.at[pl.ds(tb, TPW)], idxo_v)
        c1 = pltpu.async_copy(rows_v, out_hbm.at[idxe_v], sem)
        c2 = pltpu.async_copy(rows_v, out_hbm.at[idxo_v], sem)
        c1.wait()
        c2.wait()

    return dispatch_kernel


@functools.lru_cache(maxsize=None)
def _make_sc_combine(n_rows: int, chunk: int):
    n_per_w = n_rows // NW
    n_chunks = n_per_w // chunk
    assert n_per_w % chunk == 0 and chunk % 8 == 0

    @functools.partial(
        pl.kernel,
        mesh=_sc_mesh(),
        out_type=jax.ShapeDtypeStruct((n_rows, D), jnp.float32),
        scratch_types=[
            pltpu.VMEM((chunk,), jnp.int32),
            pltpu.VMEM((chunk, D), jnp.float32),
            pltpu.SemaphoreType.DMA,
        ],
    )
    def gather_kernel(table_hbm, idx_hbm, out_hbm, idx_v, rows_v, sem):
        wid = lax.axis_index("s") * NC + lax.axis_index("c")
        base = wid * n_per_w
        for c in range(n_chunks):
            off = base + c * chunk
            pltpu.sync_copy(idx_hbm.at[pl.ds(off, chunk)], idx_v)
            pltpu.async_copy(table_hbm.at[idx_v], rows_v, sem).wait()
            pltpu.sync_copy(rows_v, out_hbm.at[pl.ds(off, chunk)])

    return gather_kernel


def _ffn_body(blk_ref, eid_ref, x_ref, w0_ref, w1_ref, w2_ref, o_ref):
    s = pl.program_id(0)

    @pl.when(blk_ref[s] == s)
    def _():
        xb = x_ref[...]
        a = jnp.dot(xb, w0_ref[0], preferred_element_type=jnp.float32,
                    precision=lax.Precision.DEFAULT)
        b = jnp.dot(xb, w1_ref[0], preferred_element_type=jnp.float32,
                    precision=lax.Precision.DEFAULT)
        h = (a * jax.nn.sigmoid(a)) * b
        o_ref[...] = jnp.dot(h, w2_ref[0], preferred_element_type=jnp.float32,
                             precision=lax.Precision.DEFAULT)


_ffn_grid_spec = pltpu.PrefetchScalarGridSpec(
    num_scalar_prefetch=2,
    grid=(G,),
    in_specs=[
        pl.BlockSpec((B, D), lambda s, blk, eid: (blk[s], 0)),
        pl.BlockSpec((1, D, F), lambda s, blk, eid: (eid[s], 0, 0)),
        pl.BlockSpec((1, D, F), lambda s, blk, eid: (eid[s], 0, 0)),
        pl.BlockSpec((1, F, D), lambda s, blk, eid: (eid[s], 0, 0)),
    ],
    out_specs=pl.BlockSpec((B, D), lambda s, blk, eid: (blk[s], 0)),
)


def _routing_metadata(e2d):
    e_flat = e2d.reshape(P)
    oh = (e_flat[:, None] == jnp.arange(E, dtype=jnp.int32)[None, :])
    oh_b = oh.reshape(NW, P // NW, E).astype(jnp.float32)
    tril_fine = jnp.tril(jnp.ones((P // NW, P // NW), jnp.float32), k=-1)
    fine = jnp.einsum("ij,bjE->biE", tril_fine, oh_b,
                      precision=lax.Precision.HIGHEST)
    bs = oh_b.sum(axis=1)
    tril_coarse = jnp.tril(jnp.ones((NW, NW), jnp.float32), k=-1)
    coarse = tril_coarse @ bs
    rank = (fine + coarse[:, None, :]).reshape(P, E)
    rank = jnp.take_along_axis(rank, e_flat[:, None], axis=1)[:, 0]
    counts = bs.sum(axis=0)
    padded_counts = jnp.ceil(counts / B) * B
    pcsum = (jnp.tril(jnp.ones((E, E), jnp.float32)) @ padded_counts)
    pad_start = pcsum - padded_counts
    pos = (jnp.take(pad_start, e_flat) + rank).astype(jnp.int32)

    nb = (pcsum[E - 1] / B).astype(jnp.int32)
    s_ids = jnp.arange(G, dtype=jnp.int32)
    blk = jnp.minimum(s_ids, nb - 1)
    starts = (blk * B).astype(jnp.float32)
    eid = jnp.sum(pcsum[None, :] <= starts[:, None], axis=1).astype(jnp.int32)
    return pos.reshape(T, K), blk, eid


def kernel(x, w0, w1, w2, selected_experts, routing_weights):
    e2d = selected_experts.astype(jnp.int32)
    pos2d, blk, eid = _routing_metadata(e2d)

    x_padded = _make_sc_dispatch()(x, pos2d[:, 0], pos2d[:, 1])

    y = pl.pallas_call(
        _ffn_body,
        grid_spec=_ffn_grid_spec,
        out_shape=jax.ShapeDtypeStruct((P_PAD, D), jnp.float32),
    )(blk, eid, x_padded, w0, w1, w2)

    out_flat = _make_sc_combine(P, 128)(y, pos2d.reshape(P))
    out_flat = out_flat * routing_weights.reshape(P, 1)
    return out_flat.reshape(T, K, D)


# speedup vs baseline: 1.6256x; 1.1730x over previous
import functools

import jax
import jax.numpy as jnp
from jax import lax
from jax.experimental import pallas as pl
from jax.experimental.pallas import tpu as pltpu
from jax.experimental.pallas import tpu_sc as plsc

T = 2048
D = 768
F = 2048
E = 8
K = 2

B = 256
P = T * K
P_PAD = P + E * B
G = P_PAD // B

NC = 2
NS = 16
NW = NC * NS
TPW = T // NW


def _sc_mesh():
    return plsc.VectorSubcoreMesh(core_axis_name="c", subcore_axis_name="s",
                                  num_cores=NC, num_subcores=NS)


@functools.lru_cache(maxsize=None)
def _make_sc_dispatch():
    @functools.partial(
        pl.kernel,
        mesh=_sc_mesh(),
        out_type=jax.ShapeDtypeStruct((P_PAD, D), jnp.float32),
        scratch_types=[
            pltpu.VMEM((TPW,), jnp.int32),
            pltpu.VMEM((TPW,), jnp.int32),
            pltpu.VMEM((TPW, D), jnp.float32),
            pltpu.SemaphoreType.DMA,
        ],
    )
    def dispatch_kernel(x_hbm, pe_hbm, po_hbm, out_hbm, idxe_v, idxo_v,
                        rows_v, sem):
        wid = lax.axis_index("s") * NC + lax.axis_index("c")
        tb = wid * TPW
        pltpu.sync_copy(x_hbm.at[pl.ds(tb, TPW)], rows_v)
        pltpu.sync_copy(pe_hbm.at[pl.ds(tb, TPW)], idxe_v)
        pltpu.sync_copy(po_hbm.at[pl.ds(tb, TPW)], idxo_v)
        c1 = pltpu.async_copy(rows_v, out_hbm.at[idxe_v], sem)
        c2 = pltpu.async_copy(rows_v, out_hbm.at[idxo_v], sem)
        c1.wait()
        c2.wait()

    return dispatch_kernel


@functools.lru_cache(maxsize=None)
def _make_sc_combine_tkd(chunk: int):
    n_per_w = T // NW
    n_chunks = n_per_w // chunk
    assert n_per_w % chunk == 0 and chunk % 8 == 0

    @functools.partial(
        pl.kernel,
        mesh=_sc_mesh(),
        out_type=jax.ShapeDtypeStruct((T, K, D), jnp.float32),
        scratch_types=[
            pltpu.VMEM((chunk,), jnp.int32),
            pltpu.VMEM((chunk,), jnp.int32),
            pltpu.VMEM((chunk, D), jnp.float32),
            pltpu.VMEM((chunk, D), jnp.float32),
            pltpu.SemaphoreType.DMA,
        ],
    )
    def combine_kernel(table_hbm, pe_hbm, po_hbm, out_hbm, idxe_v, idxo_v,
                       rows_a, rows_b, sem):
        wid = lax.axis_index("s") * NC + lax.axis_index("c")
        base = wid * n_per_w
        for c in range(n_chunks):
            tb = base + c * chunk
            pltpu.sync_copy(pe_hbm.at[pl.ds(tb, chunk)], idxe_v)
            pltpu.sync_copy(po_hbm.at[pl.ds(tb, chunk)], idxo_v)
            c1 = pltpu.async_copy(table_hbm.at[idxe_v], rows_a, sem)
            c2 = pltpu.async_copy(table_hbm.at[idxo_v], rows_b, sem)
            c1.wait()
            c2.wait()
            pltpu.sync_copy(rows_a, out_hbm.at[pl.ds(tb, chunk), 0])
            pltpu.sync_copy(rows_b, out_hbm.at[pl.ds(tb, chunk), 1])

    return combine_kernel


@functools.lru_cache(maxsize=None)
def _make_sc_combine(n_rows: int, chunk: int):
    n_per_w = n_rows // NW
    n_chunks = n_per_w // chunk
    assert n_per_w % chunk == 0 and chunk % 8 == 0

    @functools.partial(
        pl.kernel,
        mesh=_sc_mesh(),
        out_type=jax.ShapeDtypeStruct((n_rows, D), jnp.float32),
        scratch_types=[
            pltpu.VMEM((chunk,), jnp.int32),
            pltpu.VMEM((chunk, D), jnp.float32),
            pltpu.SemaphoreType.DMA,
        ],
    )
    def gather_kernel(table_hbm, idx_hbm, out_hbm, idx_v, rows_v, sem):
        wid = lax.axis_index("s") * NC + lax.axis_index("c")
        base = wid * n_per_w
        for c in range(n_chunks):
            off = base + c * chunk
            pltpu.sync_copy(idx_hbm.at[pl.ds(off, chunk)], idx_v)
            pltpu.async_copy(table_hbm.at[idx_v], rows_v, sem).wait()
            pltpu.sync_copy(rows_v, out_hbm.at[pl.ds(off, chunk)])

    return gather_kernel


def _ffn_body(blk_ref, eid_ref, x_ref, w0_ref, w1_ref, w2_ref, o_ref):
    s = pl.program_id(0)

    @pl.when(blk_ref[s] == s)
    def _():
        xb = x_ref[...]
        a = jnp.dot(xb, w0_ref[0], preferred_element_type=jnp.float32,
                    precision=lax.Precision.DEFAULT)
        b = jnp.dot(xb, w1_ref[0], preferred_element_type=jnp.float32,
                    precision=lax.Precision.DEFAULT)
        h = (a * jax.nn.sigmoid(a)) * b
        o_ref[...] = jnp.dot(h, w2_ref[0], preferred_element_type=jnp.float32,
                             precision=lax.Precision.DEFAULT)


_ffn_grid_spec = pltpu.PrefetchScalarGridSpec(
    num_scalar_prefetch=2,
    grid=(G,),
    in_specs=[
        pl.BlockSpec((B, D), lambda s, blk, eid: (blk[s], 0)),
        pl.BlockSpec((1, D, F), lambda s, blk, eid: (eid[s], 0, 0)),
        pl.BlockSpec((1, D, F), lambda s, blk, eid: (eid[s], 0, 0)),
        pl.BlockSpec((1, F, D), lambda s, blk, eid: (eid[s], 0, 0)),
    ],
    out_specs=pl.BlockSpec((B, D), lambda s, blk, eid: (blk[s], 0)),
)


def _routing_metadata(e2d):
    e_flat = e2d.reshape(P)
    oh = (e_flat[:, None] == jnp.arange(E, dtype=jnp.int32)[None, :])
    oh_b = oh.reshape(NW, P // NW, E).astype(jnp.float32)
    tril_fine = jnp.tril(jnp.ones((P // NW, P // NW), jnp.float32), k=-1)
    fine = jnp.einsum("ij,bjE->biE", tril_fine, oh_b,
                      precision=lax.Precision.HIGHEST)
    bs = oh_b.sum(axis=1)
    tril_coarse = jnp.tril(jnp.ones((NW, NW), jnp.float32), k=-1)
    coarse = tril_coarse @ bs
    rank = (fine + coarse[:, None, :]).reshape(P, E)
    rank = jnp.take_along_axis(rank, e_flat[:, None], axis=1)[:, 0]
    counts = bs.sum(axis=0)
    padded_counts = jnp.ceil(counts / B) * B
    pcsum = (jnp.tril(jnp.ones((E, E), jnp.float32)) @ padded_counts)
    pad_start = pcsum - padded_counts
    pos = (jnp.take(pad_start, e_flat) + rank).astype(jnp.int32)

    nb = (pcsum[E - 1] / B).astype(jnp.int32)
    s_ids = jnp.arange(G, dtype=jnp.int32)
    blk = jnp.minimum(s_ids, nb - 1)
    starts = (blk * B).astype(jnp.float32)
    eid = jnp.sum(pcsum[None, :] <= starts[:, None], axis=1).astype(jnp.int32)
    return pos.reshape(T, K), blk, eid


def kernel(x, w0, w1, w2, selected_experts, routing_weights):
    e2d = selected_experts.astype(jnp.int32)
    pos2d, blk, eid = _routing_metadata(e2d)

    x_padded = _make_sc_dispatch()(x, pos2d[:, 0], pos2d[:, 1])

    y = pl.pallas_call(
        _ffn_body,
        grid_spec=_ffn_grid_spec,
        out_shape=jax.ShapeDtypeStruct((P_PAD, D), jnp.float32),
    )(blk, eid, x_padded, w0, w1, w2)

    out = _make_sc_combine_tkd(32)(y, pos2d[:, 0], pos2d[:, 1])
    return out * routing_weights[:, :, None]
